# baseline (speedup 1.0000x reference)
import functools

import jax
import jax.numpy as jnp
from jax import lax
from jax.experimental import pallas as pl
from jax.experimental.pallas import tpu as pltpu
from jax.experimental.pallas import tpu_sc as plsc

_EMB = 64
_H = 256
_NUM_LABEL = 128
_B = 4096
_L = 200

_VROWS = 1000001
_TCHUNK = 1024
_TGRID = (_VROWS + _TCHUNK - 1) // _TCHUNK
_VPAD = _TGRID * _TCHUNK

_NC, _NS = 2, 16
_NW = _NC * _NS
_BPW = _B // _NW
_NBUF = 8
_NGROUPS = _L // _NBUF
_LANES = 16


def _relayout_tc(embT):
  def body(i_ref, o_ref):
    t = i_ref[...]
    a = t.T
    h = _TCHUNK // 2
    o_ref[...] = jnp.concatenate([a[:h, :], a[h:, :]], axis=1)

  rows = _TCHUNK // 2
  return pl.pallas_call(
      body,
      grid=(_TGRID,),
      in_specs=[pl.BlockSpec((_EMB, _TCHUNK), lambda i: (0, i))],
      out_specs=pl.BlockSpec((rows, 128), lambda i: (i, 0)),
      out_shape=jax.ShapeDtypeStruct((_TGRID * rows, 128), jnp.float32),
  )(embT)


def _pool_sc(xT, emb):
  mesh = plsc.VectorSubcoreMesh(core_axis_name="c", subcore_axis_name="s")

  @functools.partial(
      pl.kernel,
      out_type=jax.ShapeDtypeStruct((_B, _EMB), jnp.float32),
      mesh=mesh,
      scratch_types=[
          pltpu.VMEM((_L, _BPW), jnp.int32),
          [pltpu.VMEM((_BPW, _EMB), jnp.float32) for _ in range(_NBUF)],
          pltpu.SemaphoreType.DMA,
      ],
      compiler_params=pltpu.CompilerParams(use_tc_tiling_on_sc=False),
  )
  def pool(xT_hbm, emb_hbm, out_hbm, idx_v, bufs, sem):
    wid = lax.axis_index("s") * _NC + lax.axis_index("c")
    base = wid * _BPW
    pltpu.sync_copy(xT_hbm.at[:, pl.ds(base, _BPW)], idx_v)

    cps = [pltpu.async_copy(emb_hbm.at[idx_v.at[j]], bufs[j], sem)
           for j in range(_NBUF)]
    for c in cps:
      c.wait()

    def group(g, carry):
      p0 = g * _NBUF
      cs = [pltpu.async_copy(emb_hbm.at[idx_v.at[p0 + j]], bufs[j], sem,
                             add=True)
            for j in range(_NBUF)]
      for c in cs:
        c.wait()
      return carry

    lax.fori_loop(1, _NGROUPS, group, 0)

    def merge_row(r, carry):
      for d in range(_EMB // _LANES):
        s = bufs[0][r, pl.ds(d * _LANES, _LANES)]
        for j in range(1, _NBUF):
          s = s + bufs[j][r, pl.ds(d * _LANES, _LANES)]
        bufs[0][r, pl.ds(d * _LANES, _LANES)] = s
      return carry

    lax.fori_loop(0, _BPW, merge_row, 0)
    pltpu.sync_copy(bufs[0], out_hbm.at[pl.ds(base, _BPW)])

  return pool(xT, emb)


def _mlp_tc(pooled, W1, b1, W2, b2):
  blk = 1024

  def body(p_ref, w1_ref, b1_ref, w2_ref, b2_ref, o_ref):
    p = p_ref[...]
    h = jnp.dot(p, w1_ref[...], preferred_element_type=jnp.float32)
    h = h * (1.0 / _L) + b1_ref[...]
    z = jnp.dot(h, w2_ref[...], preferred_element_type=jnp.float32)
    z = z + b2_ref[...]
    o_ref[...] = jax.nn.sigmoid(z)

  return pl.pallas_call(
      body,
      grid=(_B // blk,),
      in_specs=[
          pl.BlockSpec((blk, _EMB), lambda i: (i, 0)),
          pl.BlockSpec((_EMB, _H), lambda i: (0, 0)),
          pl.BlockSpec((1, _H), lambda i: (0, 0)),
          pl.BlockSpec((_H, _NUM_LABEL), lambda i: (0, 0)),
          pl.BlockSpec((1, _NUM_LABEL), lambda i: (0, 0)),
      ],
      out_specs=pl.BlockSpec((blk, _NUM_LABEL), lambda i: (i, 0)),
      out_shape=jax.ShapeDtypeStruct((_B, _NUM_LABEL), jnp.float32),
  )(pooled, W1, b1.reshape(1, _H), W2, b2.reshape(1, _NUM_LABEL))


def kernel(x, emb, W1, b1, W2, b2):
  emb_lin = _relayout_tc(emb.T).reshape(_VPAD, _EMB)
  xw = (x & -1024) | ((x & 511) << 1) | ((x >> 9) & 1)
  pooled = _pool_sc(xw.T, emb_lin)
  return _mlp_tc(pooled, W1, b1, W2, b2)


# speedup vs baseline: 2.1624x; 2.1624x over previous
import functools

import jax
import jax.numpy as jnp
from jax import lax
from jax.experimental import pallas as pl
from jax.experimental.pallas import tpu as pltpu
from jax.experimental.pallas import tpu_sc as plsc

_EMB = 64
_H = 256
_NUM_LABEL = 128
_B = 4096
_L = 200

_VROWS = 1000001
_TCHUNK = 8192
_TGRID = (_VROWS + _TCHUNK - 1) // _TCHUNK
_VPAD = _TGRID * _TCHUNK

_NC, _NS = 2, 16
_NW = _NC * _NS
_BPW = _B // _NW
_NBUF = 8
_NGROUPS = _L // _NBUF
_LANES = 16


def _relayout_tc(embT):
  def body(i_ref, o_ref):
    t = i_ref[...]
    eye = jnp.eye(_EMB, dtype=jnp.float32)
    a = jax.lax.dot_general(t, eye, (((0,), (0,)), ((), ())),
                            preferred_element_type=jnp.float32)
    h = _TCHUNK // 2
    o_ref[...] = jnp.concatenate([a[:h, :], a[h:, :]], axis=1)

  rows = _TCHUNK // 2
  return pl.pallas_call(
      body,
      grid=(_TGRID,),
      in_specs=[pl.BlockSpec((_EMB, _TCHUNK), lambda i: (0, i))],
      out_specs=pl.BlockSpec((rows, 128), lambda i: (i, 0)),
      out_shape=jax.ShapeDtypeStruct((_TGRID * rows, 128), jnp.float32),
  )(embT)


def _pool_sc(xT, emb):
  mesh = plsc.VectorSubcoreMesh(core_axis_name="c", subcore_axis_name="s")

  @functools.partial(
      pl.kernel,
      out_type=jax.ShapeDtypeStruct((_B, _EMB), jnp.float32),
      mesh=mesh,
      scratch_types=[
          pltpu.VMEM((_L, _BPW), jnp.int32),
          [pltpu.VMEM((_BPW, _EMB), jnp.float32) for _ in range(_NBUF)],
          pltpu.SemaphoreType.DMA,
      ],
      compiler_params=pltpu.CompilerParams(use_tc_tiling_on_sc=False),
  )
  def pool(xT_hbm, emb_hbm, out_hbm, idx_v, bufs, sem):
    wid = lax.axis_index("s") * _NC + lax.axis_index("c")
    base = wid * _BPW
    pltpu.sync_copy(xT_hbm.at[:, pl.ds(base, _BPW)], idx_v)

    cps = [pltpu.async_copy(emb_hbm.at[idx_v.at[j]], bufs[j], sem)
           for j in range(_NBUF)]
    for c in cps:
      c.wait()

    def group(g, carry):
      p0 = g * _NBUF
      cs = [pltpu.async_copy(emb_hbm.at[idx_v.at[p0 + j]], bufs[j], sem,
                             add=True)
            for j in range(_NBUF)]
      for c in cs:
        c.wait()
      return carry

    lax.fori_loop(1, _NGROUPS, group, 0)

    def merge_row(r, carry):
      for d in range(_EMB // _LANES):
        s = bufs[0][r, pl.ds(d * _LANES, _LANES)]
        for j in range(1, _NBUF):
          s = s + bufs[j][r, pl.ds(d * _LANES, _LANES)]
        bufs[0][r, pl.ds(d * _LANES, _LANES)] = s
      return carry

    lax.fori_loop(0, _BPW, merge_row, 0)
    pltpu.sync_copy(bufs[0], out_hbm.at[pl.ds(base, _BPW)])

  return pool(xT, emb)


def _mlp_tc(pooled, W1, b1, W2, b2):
  blk = 1024

  def body(p_ref, w1_ref, b1_ref, w2_ref, b2_ref, o_ref):
    p = p_ref[...]
    h = jnp.dot(p, w1_ref[...], preferred_element_type=jnp.float32)
    h = h * (1.0 / _L) + b1_ref[...]
    z = jnp.dot(h, w2_ref[...], preferred_element_type=jnp.float32)
    z = z + b2_ref[...]
    o_ref[...] = jax.nn.sigmoid(z)

  return pl.pallas_call(
      body,
      grid=(_B // blk,),
      in_specs=[
          pl.BlockSpec((blk, _EMB), lambda i: (i, 0)),
          pl.BlockSpec((_EMB, _H), lambda i: (0, 0)),
          pl.BlockSpec((1, _H), lambda i: (0, 0)),
          pl.BlockSpec((_H, _NUM_LABEL), lambda i: (0, 0)),
          pl.BlockSpec((1, _NUM_LABEL), lambda i: (0, 0)),
      ],
      out_specs=pl.BlockSpec((blk, _NUM_LABEL), lambda i: (i, 0)),
      out_shape=jax.ShapeDtypeStruct((_B, _NUM_LABEL), jnp.float32),
  )(pooled, W1, b1.reshape(1, _H), W2, b2.reshape(1, _NUM_LABEL))


def kernel(x, emb, W1, b1, W2, b2):
  emb_lin = _relayout_tc(emb.T).reshape(_VPAD, _EMB)
  xw = (x & -1024) | ((x & 511) << 1) | ((x >> 9) & 1)
  pooled = _pool_sc(xw.T, emb_lin)
  return _mlp_tc(pooled, W1, b1, W2, b2)


# speedup vs baseline: 2.3656x; 1.0939x over previous
import functools

import jax
import jax.numpy as jnp
from jax import lax
from jax.experimental import pallas as pl
from jax.experimental.pallas import tpu as pltpu
from jax.experimental.pallas import tpu_sc as plsc

_EMB = 64
_H = 256
_NUM_LABEL = 128
_B = 4096
_L = 200

_VROWS = 1000001
_TCHUNK = 16384
_TGRID = (_VROWS + _TCHUNK - 1) // _TCHUNK
_VPAD = _TGRID * _TCHUNK

_NC, _NS = 2, 16
_NW = _NC * _NS
_BPW = _B // _NW
_NBUF = 8
_NGROUPS = _L // _NBUF
_LANES = 16


def _relayout_tc(embT):
  def body(i_ref, o_ref):
    t = i_ref[...]
    eye = jnp.eye(_EMB, dtype=jnp.float32)
    s = _TCHUNK // 8
    a1 = t[:, :s].T
    a2 = jax.lax.dot_general(t[:, s:], eye, (((0,), (0,)), ((), ())),
                             preferred_element_type=jnp.float32)
    a = jnp.concatenate([a1, a2], axis=0)
    h = _TCHUNK // 2
    o_ref[...] = jnp.concatenate([a[:h, :], a[h:, :]], axis=1)

  rows = _TCHUNK // 2
  return pl.pallas_call(
      body,
      grid=(_TGRID,),
      in_specs=[pl.BlockSpec((_EMB, _TCHUNK), lambda i: (0, i))],
      out_specs=pl.BlockSpec((rows, 128), lambda i: (i, 0)),
      out_shape=jax.ShapeDtypeStruct((_TGRID * rows, 128), jnp.float32),
  )(embT)


def _pool_sc(xT, emb):
  mesh = plsc.VectorSubcoreMesh(core_axis_name="c", subcore_axis_name="s")

  @functools.partial(
      pl.kernel,
      out_type=jax.ShapeDtypeStruct((_B, _EMB), jnp.float32),
      mesh=mesh,
      scratch_types=[
          pltpu.VMEM((_L, _BPW), jnp.int32),
          [pltpu.VMEM((_BPW, _EMB), jnp.float32) for _ in range(_NBUF)],
          pltpu.SemaphoreType.DMA,
      ],
      compiler_params=pltpu.CompilerParams(use_tc_tiling_on_sc=False),
  )
  def pool(xT_hbm, emb_hbm, out_hbm, idx_v, bufs, sem):
    wid = lax.axis_index("s") * _NC + lax.axis_index("c")
    base = wid * _BPW
    pltpu.sync_copy(xT_hbm.at[:, pl.ds(base, _BPW)], idx_v)

    cps = [pltpu.async_copy(emb_hbm.at[idx_v.at[j]], bufs[j], sem)
           for j in range(_NBUF)]
    for c in cps:
      c.wait()

    def group(g, carry):
      p0 = g * _NBUF
      cs = [pltpu.async_copy(emb_hbm.at[idx_v.at[p0 + j]], bufs[j], sem,
                             add=True)
            for j in range(_NBUF)]
      for c in cs:
        c.wait()
      return carry

    lax.fori_loop(1, _NGROUPS, group, 0)

    def merge_row(r, carry):
      for d in range(_EMB // _LANES):
        s = bufs[0][r, pl.ds(d * _LANES, _LANES)]
        for j in range(1, _NBUF):
          s = s + bufs[j][r, pl.ds(d * _LANES, _LANES)]
        bufs[0][r, pl.ds(d * _LANES, _LANES)] = s
      return carry

    lax.fori_loop(0, _BPW, merge_row, 0)
    pltpu.sync_copy(bufs[0], out_hbm.at[pl.ds(base, _BPW)])

  return pool(xT, emb)


def _mlp_tc(pooled, W1, b1, W2, b2):
  blk = 1024

  def body(p_ref, w1_ref, b1_ref, w2_ref, b2_ref, o_ref):
    p = p_ref[...]
    h = jnp.dot(p, w1_ref[...], preferred_element_type=jnp.float32)
    h = h * (1.0 / _L) + b1_ref[...]
    z = jnp.dot(h, w2_ref[...], preferred_element_type=jnp.float32)
    z = z + b2_ref[...]
    o_ref[...] = jax.nn.sigmoid(z)

  return pl.pallas_call(
      body,
      grid=(_B // blk,),
      in_specs=[
          pl.BlockSpec((blk, _EMB), lambda i: (i, 0)),
          pl.BlockSpec((_EMB, _H), lambda i: (0, 0)),
          pl.BlockSpec((1, _H), lambda i: (0, 0)),
          pl.BlockSpec((_H, _NUM_LABEL), lambda i: (0, 0)),
          pl.BlockSpec((1, _NUM_LABEL), lambda i: (0, 0)),
      ],
      out_specs=pl.BlockSpec((blk, _NUM_LABEL), lambda i: (i, 0)),
      out_shape=jax.ShapeDtypeStruct((_B, _NUM_LABEL), jnp.float32),
  )(pooled, W1, b1.reshape(1, _H), W2, b2.reshape(1, _NUM_LABEL))


def kernel(x, emb, W1, b1, W2, b2):
  emb_lin = _relayout_tc(emb.T).reshape(_VPAD, _EMB)
  h = _TCHUNK // 2
  hbits = h.bit_length() - 1
  xw = (x & -_TCHUNK) | ((x & (h - 1)) << 1) | ((x >> hbits) & 1)
  pooled = _pool_sc(xw.T, emb_lin)
  return _mlp_tc(pooled, W1, b1, W2, b2)
